# Initial kernel scaffold; baseline (speedup 1.0000x reference)
import jax
import jax.numpy as jnp
from jax.experimental import pallas as pl


def kernel(x, edge_index, edge_attr, u, batch, W1, b1, g1, be1, W2, b2, g2, be2, W3, b3):
    raise NotImplementedError("write your pallas kernel here")


# speedup vs baseline: 2.1799x; 2.1799x over previous
import functools

import jax
import jax.numpy as jnp
from jax import lax
from jax.experimental import pallas as pl
from jax.experimental.pallas import tpu as pltpu
from jax.experimental.pallas import tpu_sc as plsc

N = 10000
E = 320000
D = 128
DE = 16
NC = 2
NS = 16
NW = NC * NS
EPW = E // NW
CH = 80
NCH = EPW // CH
G = D // 16
NP = 10240
ZR = 32
RPT = NP // NS

f32 = jnp.float32

_mesh = plsc.VectorSubcoreMesh(core_axis_name="c", subcore_axis_name="s")


@functools.partial(
    pl.kernel,
    out_type=jax.ShapeDtypeStruct((NW, 2, D), f32),
    mesh=_mesh,
    scratch_types=[
        pltpu.VMEM((CH,), jnp.int32),
        pltpu.VMEM((CH, D), f32),
        pltpu.VMEM((CH, D), f32),
        pltpu.VMEM((2, D), f32),
        pltpu.SemaphoreType.DMA,
    ],
    compiler_params=pltpu.CompilerParams(needs_layout_passes=False),
)
def _edge_stats(xa, eb, row, out, idx_v, rows_v, eb_v, acc_v, sem):
    wid = lax.axis_index("s") * NC + lax.axis_index("c")
    base = wid * EPW
    zero = jnp.zeros((16,), f32)

    def chunk(k, carry):
        b = pl.multiple_of(base + k * CH, CH)
        pltpu.sync_copy(row.at[pl.ds(b, CH)], idx_v)
        pltpu.async_copy(xa.at[idx_v], rows_v, sem).wait()
        pltpu.sync_copy(eb.at[pl.ds(b, CH)], eb_v)

        def edge(e, c):
            s = list(c[:G])
            q = list(c[G:])
            for g in range(G):
                h = rows_v[e, pl.ds(16 * g, 16)] + eb_v[e, pl.ds(16 * g, 16)]
                s[g] = s[g] + h
                q[g] = q[g] + h * h
            return tuple(s) + tuple(q)

        return lax.fori_loop(0, CH, edge, carry)

    st = lax.fori_loop(0, NCH, chunk, (zero,) * (2 * G))
    for g in range(G):
        acc_v[0, pl.ds(16 * g, 16)] = st[g]
        acc_v[1, pl.ds(16 * g, 16)] = st[G + g]
    pltpu.sync_copy(acc_v, out.at[wid])


@functools.partial(
    pl.kernel,
    out_type=[jax.ShapeDtypeStruct((NC, NP, D), f32),
              jax.ShapeDtypeStruct((NW, NP), f32)],
    mesh=_mesh,
    scratch_types=[
        pltpu.VMEM((CH,), jnp.int32),
        pltpu.VMEM((CH,), jnp.int32),
        pltpu.VMEM((CH, D), f32),
        pltpu.VMEM((CH, D), f32),
        pltpu.VMEM((CH, D), f32),
        pltpu.VMEM((2, D), f32),
        pltpu.VMEM((ZR, D), f32),
        pltpu.VMEM((NP,), f32),
        pltpu.VMEM_SHARED((NP, D), f32),
        pltpu.SemaphoreType.DMA,
    ],
    compiler_params=pltpu.CompilerParams(needs_layout_passes=False),
)
def _edge_scatter(xa, eb, row, col, aff, out, out_c,
                  idx_r, idx_c, rows_v, eb_v, vals_v, aff_v, zb, cnt_v,
                  acc_sh, sem):
    cid = lax.axis_index("c")
    sid = lax.axis_index("s")
    wid = sid * NC + cid
    base = wid * EPW
    zero = jnp.zeros((16,), f32)
    ones = jnp.ones((16,), f32)

    def zrow(r, _):
        for g in range(G):
            zb[r, pl.ds(16 * g, 16)] = zero
        return 0

    lax.fori_loop(0, ZR, zrow, 0)

    def zcnt(i, _):
        cnt_v[pl.ds(pl.multiple_of(i * 16, 16), 16)] = zero
        return 0

    lax.fori_loop(0, NP // 16, zcnt, 0)

    def zcopy(i, _):
        pltpu.sync_copy(zb, acc_sh.at[pl.ds(pl.multiple_of(sid * RPT + i * ZR, ZR), ZR)])
        return 0

    lax.fori_loop(0, RPT // ZR, zcopy, 0)

    pltpu.sync_copy(aff, aff_v)
    plsc.subcore_barrier()

    av = tuple(aff_v[0, pl.ds(16 * g, 16)] for g in range(G))
    cv = tuple(aff_v[1, pl.ds(16 * g, 16)] for g in range(G))

    def chunk(k, carry):
        b = pl.multiple_of(base + k * CH, CH)
        pltpu.sync_copy(row.at[pl.ds(b, CH)], idx_r)
        pltpu.sync_copy(col.at[pl.ds(b, CH)], idx_c)
        pltpu.async_copy(xa.at[idx_r], rows_v, sem).wait()
        pltpu.sync_copy(eb.at[pl.ds(b, CH)], eb_v)

        def edge(e, c2):
            for g in range(G):
                h = rows_v[e, pl.ds(16 * g, 16)] + eb_v[e, pl.ds(16 * g, 16)]
                vals_v[e, pl.ds(16 * g, 16)] = jnp.maximum(av[g] * h + cv[g], 0.0)
            return c2

        lax.fori_loop(0, CH, edge, 0)
        for j in range(CH // 16):
            iv = idx_c[pl.ds(16 * j, 16)]
            plsc.addupdate_scatter(cnt_v, [iv], ones)
        pltpu.sync_copy(vals_v, acc_sh.at[idx_c], add=True)
        return carry

    lax.fori_loop(0, NCH, chunk, 0)
    plsc.subcore_barrier()
    off = pl.multiple_of(sid * RPT, RPT)
    pltpu.sync_copy(acc_sh.at[pl.ds(off, RPT)], out.at[cid, pl.ds(off, RPT)])
    pltpu.sync_copy(cnt_v, out_c.at[wid])


def _xa_body(x_ref, w_ref, o_ref):
    o_ref[...] = jnp.dot(x_ref[...], w_ref[0:D, :], preferred_element_type=f32)


def _eb_body(ea_ref, w_ref, b_ref, o_ref):
    o_ref[...] = (jnp.dot(ea_ref[...], w_ref[D:D + DE, :],
                          preferred_element_type=f32) + b_ref[...])


def _aff_body(p_ref, g_ref, be_ref, o_ref):
    s = jnp.sum(p_ref[...], axis=0)
    mean = s[0:1, :] * (1.0 / E)
    var = jnp.maximum(s[1:2, :] * (1.0 / E) - mean * mean, 0.0)
    a = g_ref[...] * lax.rsqrt(var + 1e-5)
    c = be_ref[...] - mean * a
    o_ref[...] = jnp.concatenate([a, c], axis=0)


def _final_body(x_ref, s_ref, c_ref, w2_ref, b2_ref, g2_ref, be2_ref, w3_ref,
                b3_ref, o_ref):
    s = s_ref[0, 0:N, :] + s_ref[1, 0:N, :]
    cnt = jnp.sum(c_ref[...], axis=0)[0:N].reshape(N, 1)
    mean = s / jnp.maximum(cnt, 1.0)
    h = (jnp.dot(x_ref[...], w2_ref[0:D, :], preferred_element_type=f32)
         + jnp.dot(mean, w2_ref[D:2 * D, :], preferred_element_type=f32)
         + b2_ref[...])
    m = jnp.mean(h, axis=0, keepdims=True)
    v = jnp.mean((h - m) ** 2, axis=0, keepdims=True)
    hn = jnp.maximum((h - m) * lax.rsqrt(v + 1e-5) * g2_ref[...] + be2_ref[...],
                     0.0)
    o_ref[...] = jnp.maximum(
        jnp.dot(hn, w3_ref[...], preferred_element_type=f32) + b3_ref[...], 0.0)


_EB_BLK = 8000


def kernel(x, edge_index, edge_attr, u, batch,
           W1, b1, g1, be1, W2, b2, g2, be2, W3, b3):
    row = edge_index[0]
    col = edge_index[1]

    xa = pl.pallas_call(
        _xa_body,
        out_shape=jax.ShapeDtypeStruct((N, D), f32),
    )(x, W1)

    eb = pl.pallas_call(
        _eb_body,
        grid=(E // _EB_BLK,),
        in_specs=[
            pl.BlockSpec((_EB_BLK, DE), lambda i: (i, 0)),
            pl.BlockSpec((D + DE, D), lambda i: (0, 0)),
            pl.BlockSpec((1, D), lambda i: (0, 0)),
        ],
        out_specs=pl.BlockSpec((_EB_BLK, D), lambda i: (i, 0)),
        out_shape=jax.ShapeDtypeStruct((E, D), f32),
    )(edge_attr, W1, b1.reshape(1, D))

    parts = _edge_stats(xa, eb, row)

    aff = pl.pallas_call(
        _aff_body,
        out_shape=jax.ShapeDtypeStruct((2, D), f32),
    )(parts, g1.reshape(1, D), be1.reshape(1, D))

    sacc, cacc = _edge_scatter(xa, eb, row, col, aff)

    out = pl.pallas_call(
        _final_body,
        out_shape=jax.ShapeDtypeStruct((N, D), f32),
    )(x, sacc, cacc, W2, b2.reshape(1, 2 * D), g2.reshape(1, 2 * D),
      be2.reshape(1, 2 * D), W3, b3.reshape(1, D))

    return out


# speedup vs baseline: 3.5159x; 1.6129x over previous
import functools

import jax
import jax.numpy as jnp
from jax import lax
from jax.experimental import pallas as pl
from jax.experimental.pallas import tpu as pltpu
from jax.experimental.pallas import tpu_sc as plsc

N = 10000
E = 320000
D = 128
DE = 16
NC = 2
NS = 16
NW = NC * NS
EPW = E // NW
CH = 80
NCH = EPW // CH
CH2 = 40
NCH2 = EPW // CH2
G = D // 16
NP = 10240
ZR = 32
RPT = NP // NS

f32 = jnp.float32

_mesh = plsc.VectorSubcoreMesh(core_axis_name="c", subcore_axis_name="s")


@functools.partial(
    pl.kernel,
    out_type=[jax.ShapeDtypeStruct((NW, 2, D), f32),
              jax.ShapeDtypeStruct((NW, NP), f32)],
    mesh=_mesh,
    scratch_types=[
        pltpu.VMEM((EPW,), jnp.int32),
        pltpu.VMEM((EPW,), jnp.int32),
        pltpu.VMEM((CH, D), f32),
        pltpu.VMEM((CH, D), f32),
        pltpu.VMEM((CH, D), f32),
        pltpu.VMEM((CH, D), f32),
        pltpu.VMEM((2, D), f32),
        pltpu.VMEM((NP,), f32),
        pltpu.SemaphoreType.DMA,
        pltpu.SemaphoreType.DMA,
        pltpu.SemaphoreType.DMA,
        pltpu.SemaphoreType.DMA,
    ],
    compiler_params=pltpu.CompilerParams(needs_layout_passes=False),
)
def _edge_stats(xa, eb, row, col, out, out_c, idxs, idxc, rows_0, rows_1,
                eb_0, eb_1, acc_v, cnt_v, gs_0, gs_1, es_0, es_1):
    wid = lax.axis_index("s") * NC + lax.axis_index("c")
    base = pl.multiple_of(wid * EPW, EPW)
    zero = jnp.zeros((16,), f32)
    ones = jnp.ones((16,), f32)
    rows_ = (rows_0, rows_1)
    eb_ = (eb_0, eb_1)
    gs_ = (gs_0, gs_1)
    es_ = (es_0, es_1)

    def zcnt(i, _):
        cnt_v[pl.ds(pl.multiple_of(i * 16, 16), 16)] = zero
        return 0

    lax.fori_loop(0, NP // 16, zcnt, 0)

    pltpu.sync_copy(row.at[pl.ds(base, EPW)], idxs)
    pltpu.sync_copy(col.at[pl.ds(base, EPW)], idxc)

    def gather_pair(k, b):
        off = pl.multiple_of(k * CH, CH)
        src_g = xa.at[idxs.at[pl.ds(off, CH)]]
        src_e = eb.at[pl.ds(pl.multiple_of(base + off, CH), CH)]
        return (pltpu.make_async_copy(src_g, rows_[b], gs_[b]),
                pltpu.make_async_copy(src_e, eb_[b], es_[b]))

    def issue(k, b):
        for d in gather_pair(k, b):
            d.start()

    def wait(k, b):
        for d in gather_pair(k, b):
            d.wait()

    def compute(b, carry):
        rv, ev = rows_[b], eb_[b]

        def edge(e, c):
            s = list(c[:G])
            q = list(c[G:])
            for g in range(G):
                h = rv[e, pl.ds(16 * g, 16)] + ev[e, pl.ds(16 * g, 16)]
                s[g] = s[g] + h
                q[g] = q[g] + h * h
            return tuple(s) + tuple(q)

        return lax.fori_loop(0, CH, edge, carry)

    issue(0, 0)
    issue(1, 1)

    def count(k):
        koff = pl.multiple_of(k * CH, 16)
        for j in range(CH // 16):
            iv = idxc[pl.ds(koff + 16 * j, 16)]
            plsc.addupdate_scatter(cnt_v, [iv], ones)

    def dstep(i, carry):
        for b in range(2):
            k = 2 * i + b
            wait(k, b)
            carry = compute(b, carry)
            count(k)

            @pl.when(k + 2 < NCH)
            def _():
                issue(k + 2, b)

        return carry

    st = lax.fori_loop(0, (NCH - 1) // 2, dstep, (zero,) * (2 * G))
    wait(NCH - 1, 0)
    st = compute(0, st)
    count(NCH - 1)
    for g in range(G):
        acc_v[0, pl.ds(16 * g, 16)] = st[g]
        acc_v[1, pl.ds(16 * g, 16)] = st[G + g]
    pltpu.sync_copy(acc_v, out.at[wid])
    pltpu.sync_copy(cnt_v, out_c.at[wid])


@functools.partial(
    pl.kernel,
    out_type=jax.ShapeDtypeStruct((NC, NP, D), f32),
    mesh=_mesh,
    scratch_types=[
        pltpu.VMEM((CH2,), jnp.int32),
        pltpu.VMEM((CH2,), jnp.int32),
        pltpu.VMEM((CH2,), jnp.int32),
        pltpu.VMEM((CH2,), jnp.int32),
        pltpu.VMEM((CH2, D), f32),
        pltpu.VMEM((CH2, D), f32),
        pltpu.VMEM((CH2, D), f32),
        pltpu.VMEM((CH2, D), f32),
        pltpu.VMEM((CH2, D), f32),
        pltpu.VMEM((CH2, D), f32),
        pltpu.VMEM((2, D), f32),
        pltpu.VMEM((ZR, D), f32),
        pltpu.VMEM_SHARED((NP, D), f32),
        pltpu.SemaphoreType.DMA, pltpu.SemaphoreType.DMA,
        pltpu.SemaphoreType.DMA, pltpu.SemaphoreType.DMA,
        pltpu.SemaphoreType.DMA, pltpu.SemaphoreType.DMA,
        pltpu.SemaphoreType.DMA, pltpu.SemaphoreType.DMA,
        pltpu.SemaphoreType.DMA, pltpu.SemaphoreType.DMA,
    ],
    compiler_params=pltpu.CompilerParams(needs_layout_passes=False),
)
def _edge_scatter(xa, eb, row, col, aff, out,
                  gix_0, gix_1, scs_0, scs_1, rows_0, rows_1, ebv_0, ebv_1,
                  vals_0, vals_1, aff_v, zb, acc_sh,
                  gs_0, gs_1, es_0, es_1, ss_0, ss_1, is_0, is_1, js_0, js_1):
    cid = lax.axis_index("c")
    sid = lax.axis_index("s")
    base = pl.multiple_of((sid * NC + cid) * EPW, EPW)
    zero = jnp.zeros((16,), f32)
    gix_ = (gix_0, gix_1)
    scs_ = (scs_0, scs_1)
    rows_ = (rows_0, rows_1)
    ebv_ = (ebv_0, ebv_1)
    vals_ = (vals_0, vals_1)
    gs_ = (gs_0, gs_1)
    es_ = (es_0, es_1)
    ss_ = (ss_0, ss_1)
    is_ = (is_0, is_1)
    js_ = (js_0, js_1)

    def zrow(r, _):
        for g in range(G):
            zb[r, pl.ds(16 * g, 16)] = zero
        return 0

    lax.fori_loop(0, ZR, zrow, 0)

    def zcopy(i, _):
        pltpu.sync_copy(zb, acc_sh.at[pl.ds(pl.multiple_of(sid * RPT + i * ZR, ZR), ZR)])
        return 0

    lax.fori_loop(0, RPT // ZR, zcopy, 0)

    pltpu.sync_copy(aff, aff_v)
    plsc.subcore_barrier()

    av = tuple(aff_v[0, pl.ds(16 * g, 16)] for g in range(G))
    cv = tuple(aff_v[1, pl.ds(16 * g, 16)] for g in range(G))

    def gix_copy(k, b):
        off = pl.multiple_of(base + k * CH2, CH2)
        return pltpu.make_async_copy(row.at[pl.ds(off, CH2)], gix_[b], js_[b])

    def scs_copy(k, b):
        off = pl.multiple_of(base + k * CH2, CH2)
        return pltpu.make_async_copy(col.at[pl.ds(off, CH2)], scs_[b], is_[b])

    def gather_pair(k, b):
        off = pl.multiple_of(base + k * CH2, CH2)
        return (pltpu.make_async_copy(xa.at[gix_[b]], rows_[b], gs_[b]),
                pltpu.make_async_copy(eb.at[pl.ds(off, CH2)], ebv_[b], es_[b]))

    def scatter(b):
        return pltpu.make_async_copy(vals_[b], acc_sh.at[scs_[b]], ss_[b])

    gix_copy(0, 0).start()
    gix_copy(1, 1).start()
    gix_copy(0, 0).wait()
    for d in gather_pair(0, 0):
        d.start()

    def step(k, b):
        @pl.when(k >= 2)
        def _():
            scatter(b).wait()

        scs_copy(k, b).start()
        for d in gather_pair(k, b):
            d.wait()

        @pl.when(k + 2 < NCH2)
        def _():
            gix_copy(k + 2, b).start()

        rv, ev, vv = rows_[b], ebv_[b], vals_[b]

        def edge(e, c2):
            for g in range(G):
                h = rv[e, pl.ds(16 * g, 16)] + ev[e, pl.ds(16 * g, 16)]
                vv[e, pl.ds(16 * g, 16)] = jnp.maximum(av[g] * h + cv[g], 0.0)
            return c2

        lax.fori_loop(0, CH2, edge, 0)
        scs_copy(k, b).wait()
        pltpu.async_copy(vals_[b], acc_sh.at[scs_[b]], ss_[b], add=True)

        @pl.when(k + 1 < NCH2)
        def _():
            gix_copy(k + 1, 1 - b).wait()
            for d in gather_pair(k + 1, 1 - b):
                d.start()

    def dstep(i, carry):
        for b in range(2):
            step(2 * i + b, b)
        return carry

    lax.fori_loop(0, NCH2 // 2, dstep, 0)
    scatter(0).wait()
    scatter(1).wait()
    plsc.subcore_barrier()
    off = pl.multiple_of(sid * RPT, RPT)
    pltpu.sync_copy(acc_sh.at[pl.ds(off, RPT)], out.at[cid, pl.ds(off, RPT)])


def _xa_body(x_ref, w_ref, o_ref):
    o_ref[...] = jnp.dot(x_ref[...], w_ref[0:D, :], preferred_element_type=f32)


def _eb_body(ea_ref, w_ref, b_ref, o_ref):
    o_ref[...] = (jnp.dot(ea_ref[...], w_ref[D:D + DE, :],
                          preferred_element_type=f32) + b_ref[...])


def _aff_body(p_ref, g_ref, be_ref, o_ref):
    s = jnp.sum(p_ref[...], axis=0)
    mean = s[0:1, :] * (1.0 / E)
    var = jnp.maximum(s[1:2, :] * (1.0 / E) - mean * mean, 0.0)
    a = g_ref[...] * lax.rsqrt(var + 1e-5)
    c = be_ref[...] - mean * a
    o_ref[...] = jnp.concatenate([a, c], axis=0)


def _final_body(x_ref, s_ref, c_ref, w2_ref, b2_ref, g2_ref, be2_ref, w3_ref,
                b3_ref, o_ref):
    s = s_ref[0, 0:N, :] + s_ref[1, 0:N, :]
    cnt = jnp.sum(c_ref[...], axis=0)[0:N].reshape(N, 1)
    mean = s / jnp.maximum(cnt, 1.0)
    h = (jnp.dot(x_ref[...], w2_ref[0:D, :], preferred_element_type=f32)
         + jnp.dot(mean, w2_ref[D:2 * D, :], preferred_element_type=f32)
         + b2_ref[...])
    m = jnp.mean(h, axis=0, keepdims=True)
    v = jnp.mean((h - m) ** 2, axis=0, keepdims=True)
    hn = jnp.maximum((h - m) * lax.rsqrt(v + 1e-5) * g2_ref[...] + be2_ref[...],
                     0.0)
    o_ref[...] = jnp.maximum(
        jnp.dot(hn, w3_ref[...], preferred_element_type=f32) + b3_ref[...], 0.0)


_EB_BLK = 8000


def kernel(x, edge_index, edge_attr, u, batch,
           W1, b1, g1, be1, W2, b2, g2, be2, W3, b3):
    row = edge_index[0]
    col = edge_index[1]

    xa = pl.pallas_call(
        _xa_body,
        out_shape=jax.ShapeDtypeStruct((N, D), f32),
    )(x, W1)

    eb = pl.pallas_call(
        _eb_body,
        grid=(E // _EB_BLK,),
        in_specs=[
            pl.BlockSpec((_EB_BLK, DE), lambda i: (i, 0)),
            pl.BlockSpec((D + DE, D), lambda i: (0, 0)),
            pl.BlockSpec((1, D), lambda i: (0, 0)),
        ],
        out_specs=pl.BlockSpec((_EB_BLK, D), lambda i: (i, 0)),
        out_shape=jax.ShapeDtypeStruct((E, D), f32),
    )(edge_attr, W1, b1.reshape(1, D))

    parts, cacc = _edge_stats(xa, eb, row, col)

    aff = pl.pallas_call(
        _aff_body,
        out_shape=jax.ShapeDtypeStruct((2, D), f32),
    )(parts, g1.reshape(1, D), be1.reshape(1, D))

    sacc = _edge_scatter(xa, eb, row, col, aff)

    out = pl.pallas_call(
        _final_body,
        out_shape=jax.ShapeDtypeStruct((N, D), f32),
    )(x, sacc, cacc, W2, b2.reshape(1, 2 * D), g2.reshape(1, 2 * D),
      be2.reshape(1, 2 * D), W3, b3.reshape(1, D))

    return out


# speedup vs baseline: 4.7364x; 1.3471x over previous
import functools

import jax
import jax.numpy as jnp
from jax import lax
from jax.experimental import pallas as pl
from jax.experimental.pallas import tpu as pltpu
from jax.experimental.pallas import tpu_sc as plsc

N = 10000
E = 320000
D = 128
DE = 16
NC = 2
NS = 16
NW = NC * NS
EPW = E // NW
CH = 80
NCH = EPW // CH
DH = D // 2
G = D // 16
NP = 10240
ZR = 32
RPT = NP // NS

f32 = jnp.float32

_mesh = plsc.VectorSubcoreMesh(core_axis_name="c", subcore_axis_name="s")


@functools.partial(
    pl.kernel,
    out_type=[jax.ShapeDtypeStruct((NW, 2, D), f32),
              jax.ShapeDtypeStruct((NW, NP), f32),
              jax.ShapeDtypeStruct((E, DH), jnp.int32)],
    mesh=_mesh,
    scratch_types=[
        pltpu.VMEM((EPW,), jnp.int32),
        pltpu.VMEM((EPW,), jnp.int32),
        pltpu.VMEM((CH, D), f32),
        pltpu.VMEM((CH, D), f32),
        pltpu.VMEM((CH, D), f32),
        pltpu.VMEM((CH, D), f32),
        pltpu.VMEM((CH, DH), jnp.int32),
        pltpu.VMEM((CH, DH), jnp.int32),
        pltpu.VMEM((2, D), f32),
        pltpu.VMEM((NP,), f32),
        pltpu.SemaphoreType.DMA,
        pltpu.SemaphoreType.DMA,
        pltpu.SemaphoreType.DMA,
        pltpu.SemaphoreType.DMA,
        pltpu.SemaphoreType.DMA,
        pltpu.SemaphoreType.DMA,
    ],
    compiler_params=pltpu.CompilerParams(needs_layout_passes=False),
)
def _edge_stats(xa, eb, row, col, out, out_c, hb, idxs, idxc, rows_0, rows_1,
                eb_0, eb_1, hb_0, hb_1, acc_v, cnt_v,
                gs_0, gs_1, es_0, es_1, hs_0, hs_1):
    wid = lax.axis_index("s") * NC + lax.axis_index("c")
    base = pl.multiple_of(wid * EPW, EPW)
    zero = jnp.zeros((16,), f32)
    ones = jnp.ones((16,), f32)
    rows_ = (rows_0, rows_1)
    eb_ = (eb_0, eb_1)
    hb_ = (hb_0, hb_1)
    gs_ = (gs_0, gs_1)
    es_ = (es_0, es_1)
    hs_ = (hs_0, hs_1)

    def zcnt(i, _):
        cnt_v[pl.ds(pl.multiple_of(i * 16, 16), 16)] = zero
        return 0

    lax.fori_loop(0, NP // 16, zcnt, 0)

    pltpu.sync_copy(row.at[pl.ds(base, EPW)], idxs)
    pltpu.sync_copy(col.at[pl.ds(base, EPW)], idxc)

    def gather_pair(k, b):
        off = pl.multiple_of(k * CH, CH)
        src_g = xa.at[idxs.at[pl.ds(off, CH)]]
        src_e = eb.at[pl.ds(pl.multiple_of(base + off, CH), CH)]
        return (pltpu.make_async_copy(src_g, rows_[b], gs_[b]),
                pltpu.make_async_copy(src_e, eb_[b], es_[b]))

    def hb_copy(k, b):
        off = pl.multiple_of(base + k * CH, CH)
        return pltpu.make_async_copy(hb_[b], hb.at[pl.ds(off, CH)], hs_[b])

    def issue(k, b):
        for d in gather_pair(k, b):
            d.start()

    def wait(k, b):
        for d in gather_pair(k, b):
            d.wait()

    def compute(b, carry):
        rv, ev, hv = rows_[b], eb_[b], hb_[b]

        def edge(e, c):
            s = list(c[:G])
            q = list(c[G:])
            hs = []
            for g in range(G):
                h = rv[e, pl.ds(16 * g, 16)] + ev[e, pl.ds(16 * g, 16)]
                s[g] = s[g] + h
                q[g] = q[g] + h * h
                hs.append(h)
            for p in range(G // 2):
                pk = plsc.pack(hs[2 * p], hs[2 * p + 1],
                               format=plsc.PackFormat.INTERLEAVED)
                hv[e, pl.ds(16 * p, 16)] = plsc.bitcast(pk, jnp.int32)
            return tuple(s) + tuple(q)

        return lax.fori_loop(0, CH, edge, carry)

    issue(0, 0)
    issue(1, 1)

    def count(k):
        koff = pl.multiple_of(k * CH, 16)
        for j in range(CH // 16):
            iv = idxc[pl.ds(koff + 16 * j, 16)]
            plsc.addupdate_scatter(cnt_v, [iv], ones)

    def step(k, b, carry):
        @pl.when(k >= 2)
        def _():
            hb_copy(k - 2, b).wait()

        wait(k, b)
        carry = compute(b, carry)
        count(k)
        hb_copy(k, b).start()

        @pl.when(k + 2 < NCH)
        def _():
            issue(k + 2, b)

        return carry

    def dstep(i, carry):
        for b in range(2):
            carry = step(2 * i + b, b, carry)
        return carry

    st = lax.fori_loop(0, (NCH - 1) // 2, dstep, (zero,) * (2 * G))
    st = step(NCH - 1, 0, st)
    hb_copy(NCH - 2, 1).wait()
    hb_copy(NCH - 1, 0).wait()
    for g in range(G):
        acc_v[0, pl.ds(16 * g, 16)] = st[g]
        acc_v[1, pl.ds(16 * g, 16)] = st[G + g]
    pltpu.sync_copy(acc_v, out.at[wid])
    pltpu.sync_copy(cnt_v, out_c.at[wid])


@functools.partial(
    pl.kernel,
    out_type=jax.ShapeDtypeStruct((NC, NP, D), f32),
    mesh=_mesh,
    scratch_types=[
        pltpu.VMEM((CH,), jnp.int32),
        pltpu.VMEM((CH,), jnp.int32),
        pltpu.VMEM((CH, DH), jnp.int32),
        pltpu.VMEM((CH, DH), jnp.int32),
        pltpu.VMEM((CH, D), f32),
        pltpu.VMEM((CH, D), f32),
        pltpu.VMEM((2, D), f32),
        pltpu.VMEM((ZR, D), f32),
        pltpu.VMEM_SHARED((NP, D), f32),
        pltpu.SemaphoreType.DMA, pltpu.SemaphoreType.DMA,
        pltpu.SemaphoreType.DMA, pltpu.SemaphoreType.DMA,
        pltpu.SemaphoreType.DMA, pltpu.SemaphoreType.DMA,
    ],
    compiler_params=pltpu.CompilerParams(needs_layout_passes=False),
)
def _edge_scatter(hb, col, aff, out,
                  scs_0, scs_1, hv_0, hv_1, vals_0, vals_1, aff_v, zb, acc_sh,
                  hs_0, hs_1, ss_0, ss_1, is_0, is_1):
    cid = lax.axis_index("c")
    sid = lax.axis_index("s")
    base = pl.multiple_of((sid * NC + cid) * EPW, EPW)
    zero = jnp.zeros((16,), f32)
    scs_ = (scs_0, scs_1)
    hv_ = (hv_0, hv_1)
    vals_ = (vals_0, vals_1)
    hs_ = (hs_0, hs_1)
    ss_ = (ss_0, ss_1)
    is_ = (is_0, is_1)

    def zrow(r, _):
        for g in range(G):
            zb[r, pl.ds(16 * g, 16)] = zero
        return 0

    lax.fori_loop(0, ZR, zrow, 0)

    def zcopy(i, _):
        pltpu.sync_copy(zb, acc_sh.at[pl.ds(pl.multiple_of(sid * RPT + i * ZR, ZR), ZR)])
        return 0

    lax.fori_loop(0, RPT // ZR, zcopy, 0)

    pltpu.sync_copy(aff, aff_v)
    plsc.subcore_barrier()

    av = tuple(aff_v[0, pl.ds(16 * g, 16)] for g in range(G))
    cv = tuple(aff_v[1, pl.ds(16 * g, 16)] for g in range(G))

    def scs_copy(k, b):
        off = pl.multiple_of(base + k * CH, CH)
        return pltpu.make_async_copy(col.at[pl.ds(off, CH)], scs_[b], is_[b])

    def hb_read(k, b):
        off = pl.multiple_of(base + k * CH, CH)
        return pltpu.make_async_copy(hb.at[pl.ds(off, CH)], hv_[b], hs_[b])

    def scatter(b):
        return pltpu.make_async_copy(vals_[b], acc_sh.at[scs_[b]], ss_[b])

    hb_read(0, 0).start()
    hb_read(1, 1).start()

    def step(k, b):
        @pl.when(k >= 2)
        def _():
            scatter(b).wait()

        scs_copy(k, b).start()
        hb_read(k, b).wait()

        hv, vv = hv_[b], vals_[b]

        def edge(e, c2):
            for p in range(G // 2):
                w = hv[e, pl.ds(16 * p, 16)]
                h0, h1 = plsc.unpack(plsc.bitcast(w, jnp.bfloat16),
                                     format=plsc.PackFormat.INTERLEAVED)
                g = 2 * p
                vv[e, pl.ds(16 * g, 16)] = jnp.maximum(av[g] * h0 + cv[g], 0.0)
                vv[e, pl.ds(16 * (g + 1), 16)] = jnp.maximum(
                    av[g + 1] * h1 + cv[g + 1], 0.0)
            return c2

        lax.fori_loop(0, CH, edge, 0)

        @pl.when(k + 2 < NCH)
        def _():
            hb_read(k + 2, b).start()

        scs_copy(k, b).wait()
        pltpu.async_copy(vals_[b], acc_sh.at[scs_[b]], ss_[b], add=True)

    def dstep(i, carry):
        for b in range(2):
            step(2 * i + b, b)
        return carry

    lax.fori_loop(0, (NCH - 1) // 2, dstep, 0)
    step(NCH - 1, 0)
    scatter(0).wait()
    scatter(1).wait()
    plsc.subcore_barrier()
    off = pl.multiple_of(sid * RPT, RPT)
    pltpu.sync_copy(acc_sh.at[pl.ds(off, RPT)], out.at[cid, pl.ds(off, RPT)])


def _xa_body(x_ref, w_ref, o_ref):
    o_ref[...] = jnp.dot(x_ref[...], w_ref[0:D, :], preferred_element_type=f32)


def _eb_body(ea_ref, w_ref, b_ref, o_ref):
    o_ref[...] = (jnp.dot(ea_ref[...], w_ref[D:D + DE, :],
                          preferred_element_type=f32) + b_ref[...])


def _aff_body(p_ref, g_ref, be_ref, o_ref):
    s = jnp.sum(p_ref[...], axis=0)
    mean = s[0:1, :] * (1.0 / E)
    var = jnp.maximum(s[1:2, :] * (1.0 / E) - mean * mean, 0.0)
    a = g_ref[...] * lax.rsqrt(var + 1e-5)
    c = be_ref[...] - mean * a
    o_ref[...] = jnp.concatenate([a, c], axis=0)


def _final_body(x_ref, s_ref, c_ref, w2_ref, b2_ref, g2_ref, be2_ref, w3_ref,
                b3_ref, o_ref):
    s = s_ref[0, 0:N, :] + s_ref[1, 0:N, :]
    cnt = jnp.sum(c_ref[...], axis=0)[0:N].reshape(N, 1)
    mean = s / jnp.maximum(cnt, 1.0)
    h = (jnp.dot(x_ref[...], w2_ref[0:D, :], preferred_element_type=f32)
         + jnp.dot(mean, w2_ref[D:2 * D, :], preferred_element_type=f32)
         + b2_ref[...])
    m = jnp.mean(h, axis=0, keepdims=True)
    v = jnp.mean((h - m) ** 2, axis=0, keepdims=True)
    hn = jnp.maximum((h - m) * lax.rsqrt(v + 1e-5) * g2_ref[...] + be2_ref[...],
                     0.0)
    o_ref[...] = jnp.maximum(
        jnp.dot(hn, w3_ref[...], preferred_element_type=f32) + b3_ref[...], 0.0)


_EB_BLK = 8000


def kernel(x, edge_index, edge_attr, u, batch,
           W1, b1, g1, be1, W2, b2, g2, be2, W3, b3):
    row = edge_index[0]
    col = edge_index[1]

    xa = pl.pallas_call(
        _xa_body,
        out_shape=jax.ShapeDtypeStruct((N, D), f32),
    )(x, W1)

    eb = pl.pallas_call(
        _eb_body,
        grid=(E // _EB_BLK,),
        in_specs=[
            pl.BlockSpec((_EB_BLK, DE), lambda i: (i, 0)),
            pl.BlockSpec((D + DE, D), lambda i: (0, 0)),
            pl.BlockSpec((1, D), lambda i: (0, 0)),
        ],
        out_specs=pl.BlockSpec((_EB_BLK, D), lambda i: (i, 0)),
        out_shape=jax.ShapeDtypeStruct((E, D), f32),
    )(edge_attr, W1, b1.reshape(1, D))

    parts, cacc, hb = _edge_stats(xa, eb, row, col)

    aff = pl.pallas_call(
        _aff_body,
        out_shape=jax.ShapeDtypeStruct((2, D), f32),
    )(parts, g1.reshape(1, D), be1.reshape(1, D))

    sacc = _edge_scatter(hb, col, aff)

    out = pl.pallas_call(
        _final_body,
        out_shape=jax.ShapeDtypeStruct((N, D), f32),
    )(x, sacc, cacc, W2, b2.reshape(1, 2 * D), g2.reshape(1, 2 * D),
      be2.reshape(1, 2 * D), W3, b3.reshape(1, D))

    return out
